# Initial kernel scaffold; baseline (speedup 1.0000x reference)
import jax
import jax.numpy as jnp
from jax.experimental import pallas as pl


def kernel(xc, yc, xt, params, senders, receivers):
    raise NotImplementedError("write your pallas kernel here")


# speedup vs baseline: 17.9414x; 17.9414x over previous
import functools

import jax
import jax.numpy as jnp
from jax import lax
from jax.experimental import pallas as pl
from jax.experimental.pallas import tpu as pltpu
from jax.experimental.pallas import tpu_sc as plsc

F32 = jnp.float32
N_NODES = 10000
N_EDGES = 160000
DIM_H = 64
IN_DIM = 66
PAD = 128
EC = 8000
NCH = 2000
SC_NC = 2
SC_NS = 16
SC_NW = SC_NC * SC_NS
BPW = N_EDGES // SC_NW
CH = 1000
CHS = 200


def _ln(x, g, b):
    m = jnp.mean(x, axis=-1, keepdims=True)
    v = jnp.mean((x - m) ** 2, axis=-1, keepdims=True)
    return (x - m) * lax.rsqrt(v + 1e-5) * g + b


def _full_spec(shape):
    nd = len(shape)
    return pl.BlockSpec(shape, lambda *_: (0,) * nd)


def _encode_body(xc_ref, yc_ref, pos_ref, w1, b1, w2, b2, w3, b3, nodes_ref):
    xc = xc_ref[0]
    yc = yc_ref[0]
    pos = pos_ref[...]
    h = jnp.concatenate([xc, yc], axis=-1)
    h = jax.nn.relu(jnp.dot(h, w1[...], preferred_element_type=F32) + b1[...])
    h = jax.nn.relu(jnp.dot(h, w2[...], preferred_element_type=F32) + b2[...])
    emb = jnp.dot(h, w3[...], preferred_element_type=F32) + b3[...]
    d2 = (jnp.sum(xc * xc, axis=-1, keepdims=True)
          - 2.0 * lax.dot_general(xc, pos, (((1,), (1,)), ((), ())),
                                  preferred_element_type=F32)
          + jnp.sum(pos * pos, axis=-1)[None, :])
    s = jax.nn.softmax(-d2, axis=-1)
    lat = lax.dot_general(s, emb, (((0,), (0,)), ((), ())),
                          preferred_element_type=F32)
    nodes_ref[0] = jnp.concatenate(
        [pos, lat, jnp.zeros((N_NODES, PAD - IN_DIM), F32)], axis=-1)


def _encode(xc, yc, pos, enc):
    (w1, b1), (w2, b2), (w3, b3) = enc
    B, M, _ = xc.shape
    return pl.pallas_call(
        _encode_body,
        grid=(B,),
        in_specs=[
            pl.BlockSpec((1, M, 2), lambda b: (b, 0, 0)),
            pl.BlockSpec((1, M, 1), lambda b: (b, 0, 0)),
            _full_spec(pos.shape),
            _full_spec(w1.shape), _full_spec(b1.shape),
            _full_spec(w2.shape), _full_spec(b2.shape),
            _full_spec(w3.shape), _full_spec(b3.shape),
        ],
        out_specs=pl.BlockSpec((1, N_NODES, PAD), lambda b: (b, 0, 0)),
        out_shape=jax.ShapeDtypeStruct((B, N_NODES, PAD), F32),
    )(xc, yc, pos, w1, b1, w2, b2, w3, b3)


def _sc_gather(table, idx):
    @functools.partial(
        pl.kernel,
        out_type=jax.ShapeDtypeStruct((N_EDGES, PAD), F32),
        mesh=plsc.VectorSubcoreMesh(core_axis_name="c", subcore_axis_name="s"),
        scratch_types=[
            pltpu.VMEM((CH,), jnp.int32),
            pltpu.VMEM((CH, PAD), F32),
            pltpu.SemaphoreType.DMA,
        ],
    )
    def k(table_hbm, idx_hbm, out_hbm, idx_v, rows_v, sem):
        wid = lax.axis_index("s") * SC_NC + lax.axis_index("c")
        base = wid * BPW
        for j in range(BPW // CH):
            off = base + j * CH
            pltpu.sync_copy(idx_hbm.at[pl.ds(off, CH)], idx_v)
            pltpu.async_copy(table_hbm.at[idx_v], rows_v, sem).wait()
            pltpu.sync_copy(rows_v, out_hbm.at[pl.ds(off, CH)])

    return k(table, idx)


def _sc_scatter(msgs, idx, zeros):
    @functools.partial(
        pl.kernel,
        out_type=jax.ShapeDtypeStruct((SC_NC, N_NODES, PAD), F32),
        mesh=plsc.VectorSubcoreMesh(core_axis_name="c", subcore_axis_name="s"),
        scratch_types=[
            pltpu.VMEM((CHS,), jnp.int32),
            pltpu.VMEM((CHS, PAD), F32),
            pltpu.VMEM_SHARED((N_NODES, PAD), F32),
        ],
    )
    def k(msg_hbm, idx_hbm, zero_hbm, out_hbm, idx_v, buf_v, acc_sh):
        cid = lax.axis_index("c")
        sid = lax.axis_index("s")
        wid = sid * SC_NC + cid
        base = wid * BPW

        @pl.when(sid == 0)
        def _():
            pltpu.sync_copy(zero_hbm, acc_sh)

        plsc.subcore_barrier()

        @pl.loop(0, BPW // CHS)
        def _(j):
            off = base + j * CHS
            pltpu.sync_copy(idx_hbm.at[pl.ds(off, CHS)], idx_v)
            pltpu.sync_copy(msg_hbm.at[pl.ds(off, CHS)], buf_v)
            pltpu.sync_copy(buf_v, acc_sh.at[idx_v], add=True)

        plsc.subcore_barrier()

        @pl.when(sid == 0)
        def _():
            pltpu.sync_copy(acc_sh, out_hbm.at[cid])

    return k(msgs, idx, zeros)


def _msg_body(gr_ref, gs_ref, wr, ws, bm, g1, b1, out_ref):
    m = (jnp.dot(gr_ref[0], wr[...], preferred_element_type=F32)
         + jnp.dot(gs_ref[0], ws[...], preferred_element_type=F32)
         + bm[...])
    m = _ln(m, g1[...], b1[...])
    out_ref[0] = jnp.concatenate(
        [m, jnp.zeros((EC, PAD - IN_DIM), F32)], axis=-1)


def _messages(gr, gs, wr, ws, bm, g1, b1):
    B = gr.shape[0]
    return pl.pallas_call(
        _msg_body,
        grid=(B, N_EDGES // EC),
        in_specs=[
            pl.BlockSpec((1, EC, PAD), lambda b, e: (b, e, 0)),
            pl.BlockSpec((1, EC, PAD), lambda b, e: (b, e, 0)),
            _full_spec(wr.shape), _full_spec(ws.shape), _full_spec(bm.shape),
            _full_spec(g1.shape), _full_spec(b1.shape),
        ],
        out_specs=pl.BlockSpec((1, EC, PAD), lambda b, e: (b, e, 0)),
        out_shape=jax.ShapeDtypeStruct((B, N_EDGES, PAD), F32),
    )(gr, gs, wr, ws, bm, g1, b1)


def _upd_body(nodes_ref, inbox_ref, pos_ref, wr, ws, bn, g2, b2,
              nodes_out, lat_out):
    nodes = nodes_ref[0]
    inbox = inbox_ref[0, 0] + inbox_ref[0, 1]
    h = (jnp.dot(nodes, wr[...], preferred_element_type=F32)
         + jnp.dot(inbox, ws[...], preferred_element_type=F32)
         + bn[...])
    h = _ln(h, g2[...], b2[...])
    lat_out[0] = h
    nodes_out[0] = jnp.concatenate(
        [pos_ref[...], h, jnp.zeros((NCH, PAD - IN_DIM), F32)], axis=-1)


def _update(nodes, inbox, pos, wr, ws, bn, g2, b2):
    B = nodes.shape[0]
    return pl.pallas_call(
        _upd_body,
        grid=(B, N_NODES // NCH),
        in_specs=[
            pl.BlockSpec((1, NCH, PAD), lambda b, n: (b, n, 0)),
            pl.BlockSpec((1, SC_NC, NCH, PAD), lambda b, n: (b, 0, n, 0)),
            pl.BlockSpec((NCH, 2), lambda b, n: (n, 0)),
            _full_spec(wr.shape), _full_spec(ws.shape), _full_spec(bn.shape),
            _full_spec(g2.shape), _full_spec(b2.shape),
        ],
        out_specs=[
            pl.BlockSpec((1, NCH, PAD), lambda b, n: (b, n, 0)),
            pl.BlockSpec((1, NCH, DIM_H), lambda b, n: (b, n, 0)),
        ],
        out_shape=[
            jax.ShapeDtypeStruct((B, N_NODES, PAD), F32),
            jax.ShapeDtypeStruct((B, N_NODES, DIM_H), F32),
        ],
    )(nodes, inbox, pos, wr, ws, bn, g2, b2)


def _dec_body(xt_ref, pos_ref, lat_ref,
              qw1, qb1, qw2, qb2, qw3, qb3,
              dw1, db1, dw2, db2, dw3, db3, out_ref):
    xt = xt_ref[0]
    pos = pos_ref[...]
    q = jax.nn.relu(jnp.dot(xt, qw1[...], preferred_element_type=F32) + qb1[...])
    q = jax.nn.relu(jnp.dot(q, qw2[...], preferred_element_type=F32) + qb2[...])
    q = jnp.dot(q, qw3[...], preferred_element_type=F32) + qb3[...]
    d2 = (jnp.sum(xt * xt, axis=-1, keepdims=True)
          - 2.0 * lax.dot_general(xt, pos, (((1,), (1,)), ((), ())),
                                  preferred_element_type=F32)
          + jnp.sum(pos * pos, axis=-1)[None, :])
    s = jax.nn.softmax(-d2, axis=-1)
    z = jnp.dot(s, lat_ref[0], preferred_element_type=F32)
    h = jnp.concatenate([z, q], axis=-1)
    h = jax.nn.relu(jnp.dot(h, dw1[...], preferred_element_type=F32) + db1[...])
    h = jax.nn.relu(jnp.dot(h, dw2[...], preferred_element_type=F32) + db2[...])
    out_ref[0] = jnp.dot(h, dw3[...], preferred_element_type=F32) + db3[...]


def _decode(xt, pos, lat, qenc, dec):
    (qw1, qb1), (qw2, qb2), (qw3, qb3) = qenc
    (dw1, db1), (dw2, db2), (dw3, db3) = dec
    B, M, _ = xt.shape
    return pl.pallas_call(
        _dec_body,
        grid=(B,),
        in_specs=[
            pl.BlockSpec((1, M, 2), lambda b: (b, 0, 0)),
            _full_spec(pos.shape),
            pl.BlockSpec((1, N_NODES, DIM_H), lambda b: (b, 0, 0)),
            _full_spec(qw1.shape), _full_spec(qb1.shape),
            _full_spec(qw2.shape), _full_spec(qb2.shape),
            _full_spec(qw3.shape), _full_spec(qb3.shape),
            _full_spec(dw1.shape), _full_spec(db1.shape),
            _full_spec(dw2.shape), _full_spec(db2.shape),
            _full_spec(dw3.shape), _full_spec(db3.shape),
        ],
        out_specs=pl.BlockSpec((1, M, 1), lambda b: (b, 0, 0)),
        out_shape=jax.ShapeDtypeStruct((B, M, 1), F32),
    )(xt, pos, lat, qw1, qb1, qw2, qb2, qw3, qb3,
      dw1, db1, dw2, db2, dw3, db3)


def _pad_rows(w):
    return jnp.concatenate(
        [w, jnp.zeros((PAD - IN_DIM, w.shape[1]), F32)], axis=0)


def kernel(xc, yc, xt, params, senders, receivers):
    pos = params['pos']
    blk = params['block']
    B = xc.shape[0]

    wm_r = _pad_rows(blk['Wm'][:IN_DIM])
    wm_s = _pad_rows(blk['Wm'][IN_DIM:])
    wn_r = _pad_rows(blk['Wn'][:IN_DIM])
    wn_s = _pad_rows(blk['Wn'][IN_DIM:])

    nodes = _encode(xc, yc, pos, params['enc'])
    zeros = jnp.zeros((N_NODES, PAD), F32)

    lat = None
    for _ in range(3):
        gr = jnp.stack([_sc_gather(nodes[b], receivers) for b in range(B)])
        gs = jnp.stack([_sc_gather(nodes[b], senders) for b in range(B)])
        msgs = _messages(gr, gs, wm_r, wm_s, blk['bm'], blk['g1'], blk['b1'])
        inbox = jnp.stack(
            [_sc_scatter(msgs[b], receivers, zeros) for b in range(B)])
        nodes, lat = _update(nodes, inbox, pos, wn_r, wn_s,
                             blk['bn'], blk['g2'], blk['b2'])

    return _decode(xt, pos, lat, params['qenc'], params['dec'])


# speedup vs baseline: 21.1954x; 1.1814x over previous
import functools

import jax
import jax.numpy as jnp
from jax import lax
from jax.experimental import pallas as pl
from jax.experimental.pallas import tpu as pltpu
from jax.experimental.pallas import tpu_sc as plsc

F32 = jnp.float32
N_NODES = 10000
N_EDGES = 160000
DIM_H = 64
IN_DIM = 66
PAD = 128
EC = 8000
NCH = 2000
SC_NC = 2
SC_NS = 16
SC_NW = SC_NC * SC_NS
BPW = N_EDGES // SC_NW
CH = 1000
CHS = 200


def _ln(x, g, b):
    m = jnp.mean(x, axis=-1, keepdims=True)
    v = jnp.mean((x - m) ** 2, axis=-1, keepdims=True)
    return (x - m) * lax.rsqrt(v + 1e-5) * g + b


def _full_spec(shape):
    nd = len(shape)
    return pl.BlockSpec(shape, lambda *_: (0,) * nd)


def _encode_body(xc_ref, yc_ref, pos_ref, w1, b1, w2, b2, w3, b3, nodes_ref):
    xc = xc_ref[0]
    yc = yc_ref[0]
    pos = pos_ref[...]
    h = jnp.concatenate([xc, yc], axis=-1)
    h = jax.nn.relu(jnp.dot(h, w1[...], preferred_element_type=F32) + b1[...])
    h = jax.nn.relu(jnp.dot(h, w2[...], preferred_element_type=F32) + b2[...])
    emb = jnp.dot(h, w3[...], preferred_element_type=F32) + b3[...]
    d2 = (jnp.sum(xc * xc, axis=-1, keepdims=True)
          - 2.0 * lax.dot_general(xc, pos, (((1,), (1,)), ((), ())),
                                  preferred_element_type=F32)
          + jnp.sum(pos * pos, axis=-1)[None, :])
    s = jax.nn.softmax(-d2, axis=-1)
    lat = lax.dot_general(s, emb, (((0,), (0,)), ((), ())),
                          preferred_element_type=F32)
    nodes_ref[0] = jnp.concatenate(
        [pos, lat, jnp.zeros((N_NODES, PAD - IN_DIM), F32)], axis=-1)


def _encode(xc, yc, pos, enc):
    (w1, b1), (w2, b2), (w3, b3) = enc
    B, M, _ = xc.shape
    return pl.pallas_call(
        _encode_body,
        grid=(B,),
        in_specs=[
            pl.BlockSpec((1, M, 2), lambda b: (b, 0, 0)),
            pl.BlockSpec((1, M, 1), lambda b: (b, 0, 0)),
            _full_spec(pos.shape),
            _full_spec(w1.shape), _full_spec(b1.shape),
            _full_spec(w2.shape), _full_spec(b2.shape),
            _full_spec(w3.shape), _full_spec(b3.shape),
        ],
        out_specs=pl.BlockSpec((1, N_NODES, PAD), lambda b: (b, 0, 0)),
        out_shape=jax.ShapeDtypeStruct((B, N_NODES, PAD), F32),
    )(xc, yc, pos, w1, b1, w2, b2, w3, b3)


def _sc_gather2(nodes, ridx, sidx):
    B = nodes.shape[0]

    @functools.partial(
        pl.kernel,
        out_type=(jax.ShapeDtypeStruct((B, N_EDGES, PAD), F32),
                  jax.ShapeDtypeStruct((B, N_EDGES, PAD), F32)),
        mesh=plsc.VectorSubcoreMesh(core_axis_name="c", subcore_axis_name="s"),
        scratch_types=[
            pltpu.VMEM((CHS,), jnp.int32),
            pltpu.VMEM((CHS,), jnp.int32),
            pltpu.VMEM((CHS, PAD), F32),
            pltpu.VMEM((CHS, PAD), F32),
            pltpu.SemaphoreType.DMA,
            pltpu.SemaphoreType.DMA,
        ],
    )
    def k(nodes_hbm, ridx_hbm, sidx_hbm, gr_hbm, gs_hbm,
          ri_v, si_v, rr_v, sr_v, sem_r, sem_s):
        wid = lax.axis_index("s") * SC_NC + lax.axis_index("c")
        base = wid * BPW
        for b in range(B):
            @pl.loop(0, BPW // CHS)
            def _(j, b=b):
                off = base + j * CHS
                pltpu.sync_copy(ridx_hbm.at[pl.ds(off, CHS)], ri_v)
                pltpu.sync_copy(sidx_hbm.at[pl.ds(off, CHS)], si_v)
                cr = pltpu.async_copy(nodes_hbm.at[b].at[ri_v], rr_v, sem_r)
                cs = pltpu.async_copy(nodes_hbm.at[b].at[si_v], sr_v, sem_s)
                cr.wait()
                cs.wait()
                pltpu.sync_copy(rr_v, gr_hbm.at[b].at[pl.ds(off, CHS)])
                pltpu.sync_copy(sr_v, gs_hbm.at[b].at[pl.ds(off, CHS)])

    return k(nodes, ridx, sidx)


def _sc_scatter(msgs, idx, zeros):
    @functools.partial(
        pl.kernel,
        out_type=jax.ShapeDtypeStruct((SC_NC, N_NODES, PAD), F32),
        mesh=plsc.VectorSubcoreMesh(core_axis_name="c", subcore_axis_name="s"),
        scratch_types=[
            pltpu.VMEM((CHS,), jnp.int32),
            pltpu.VMEM((CHS, PAD), F32),
            pltpu.VMEM_SHARED((N_NODES, PAD), F32),
        ],
    )
    def k(msg_hbm, idx_hbm, zero_hbm, out_hbm, idx_v, buf_v, acc_sh):
        cid = lax.axis_index("c")
        sid = lax.axis_index("s")
        wid = sid * SC_NC + cid
        base = wid * BPW

        @pl.when(sid == 0)
        def _():
            pltpu.sync_copy(zero_hbm, acc_sh)

        plsc.subcore_barrier()

        @pl.loop(0, BPW // CHS)
        def _(j):
            off = base + j * CHS
            pltpu.sync_copy(idx_hbm.at[pl.ds(off, CHS)], idx_v)
            pltpu.sync_copy(msg_hbm.at[pl.ds(off, CHS)], buf_v)
            pltpu.sync_copy(buf_v, acc_sh.at[idx_v], add=True)

        plsc.subcore_barrier()

        @pl.when(sid == 0)
        def _():
            pltpu.sync_copy(acc_sh, out_hbm.at[cid])

    return k(msgs, idx, zeros)


def _msg_body(gr_ref, gs_ref, wr, ws, bm, g1, b1, out_ref):
    m = (jnp.dot(gr_ref[0], wr[...], preferred_element_type=F32)
         + jnp.dot(gs_ref[0], ws[...], preferred_element_type=F32)
         + bm[...])
    m = _ln(m, g1[...], b1[...])
    out_ref[0] = jnp.concatenate(
        [m, jnp.zeros((EC, PAD - IN_DIM), F32)], axis=-1)


def _messages(gr, gs, wr, ws, bm, g1, b1):
    B = gr.shape[0]
    return pl.pallas_call(
        _msg_body,
        grid=(B, N_EDGES // EC),
        in_specs=[
            pl.BlockSpec((1, EC, PAD), lambda b, e: (b, e, 0)),
            pl.BlockSpec((1, EC, PAD), lambda b, e: (b, e, 0)),
            _full_spec(wr.shape), _full_spec(ws.shape), _full_spec(bm.shape),
            _full_spec(g1.shape), _full_spec(b1.shape),
        ],
        out_specs=pl.BlockSpec((1, EC, PAD), lambda b, e: (b, e, 0)),
        out_shape=jax.ShapeDtypeStruct((B, N_EDGES, PAD), F32),
    )(gr, gs, wr, ws, bm, g1, b1)


def _upd_body(nodes_ref, inbox_ref, pos_ref, wr, ws, bn, g2, b2,
              nodes_out, lat_out):
    nodes = nodes_ref[0]
    inbox = inbox_ref[0, 0] + inbox_ref[0, 1]
    h = (jnp.dot(nodes, wr[...], preferred_element_type=F32)
         + jnp.dot(inbox, ws[...], preferred_element_type=F32)
         + bn[...])
    h = _ln(h, g2[...], b2[...])
    lat_out[0] = h
    nodes_out[0] = jnp.concatenate(
        [pos_ref[...], h, jnp.zeros((NCH, PAD - IN_DIM), F32)], axis=-1)


def _update(nodes, inbox, pos, wr, ws, bn, g2, b2):
    B = nodes.shape[0]
    return pl.pallas_call(
        _upd_body,
        grid=(B, N_NODES // NCH),
        in_specs=[
            pl.BlockSpec((1, NCH, PAD), lambda b, n: (b, n, 0)),
            pl.BlockSpec((1, SC_NC, NCH, PAD), lambda b, n: (b, 0, n, 0)),
            pl.BlockSpec((NCH, 2), lambda b, n: (n, 0)),
            _full_spec(wr.shape), _full_spec(ws.shape), _full_spec(bn.shape),
            _full_spec(g2.shape), _full_spec(b2.shape),
        ],
        out_specs=[
            pl.BlockSpec((1, NCH, PAD), lambda b, n: (b, n, 0)),
            pl.BlockSpec((1, NCH, DIM_H), lambda b, n: (b, n, 0)),
        ],
        out_shape=[
            jax.ShapeDtypeStruct((B, N_NODES, PAD), F32),
            jax.ShapeDtypeStruct((B, N_NODES, DIM_H), F32),
        ],
    )(nodes, inbox, pos, wr, ws, bn, g2, b2)


def _dec_body(xt_ref, pos_ref, lat_ref,
              qw1, qb1, qw2, qb2, qw3, qb3,
              dw1, db1, dw2, db2, dw3, db3, out_ref):
    xt = xt_ref[0]
    pos = pos_ref[...]
    q = jax.nn.relu(jnp.dot(xt, qw1[...], preferred_element_type=F32) + qb1[...])
    q = jax.nn.relu(jnp.dot(q, qw2[...], preferred_element_type=F32) + qb2[...])
    q = jnp.dot(q, qw3[...], preferred_element_type=F32) + qb3[...]
    d2 = (jnp.sum(xt * xt, axis=-1, keepdims=True)
          - 2.0 * lax.dot_general(xt, pos, (((1,), (1,)), ((), ())),
                                  preferred_element_type=F32)
          + jnp.sum(pos * pos, axis=-1)[None, :])
    s = jax.nn.softmax(-d2, axis=-1)
    z = jnp.dot(s, lat_ref[0], preferred_element_type=F32)
    h = jnp.concatenate([z, q], axis=-1)
    h = jax.nn.relu(jnp.dot(h, dw1[...], preferred_element_type=F32) + db1[...])
    h = jax.nn.relu(jnp.dot(h, dw2[...], preferred_element_type=F32) + db2[...])
    out_ref[0] = jnp.dot(h, dw3[...], preferred_element_type=F32) + db3[...]


def _decode(xt, pos, lat, qenc, dec):
    (qw1, qb1), (qw2, qb2), (qw3, qb3) = qenc
    (dw1, db1), (dw2, db2), (dw3, db3) = dec
    B, M, _ = xt.shape
    return pl.pallas_call(
        _dec_body,
        grid=(B,),
        in_specs=[
            pl.BlockSpec((1, M, 2), lambda b: (b, 0, 0)),
            _full_spec(pos.shape),
            pl.BlockSpec((1, N_NODES, DIM_H), lambda b: (b, 0, 0)),
            _full_spec(qw1.shape), _full_spec(qb1.shape),
            _full_spec(qw2.shape), _full_spec(qb2.shape),
            _full_spec(qw3.shape), _full_spec(qb3.shape),
            _full_spec(dw1.shape), _full_spec(db1.shape),
            _full_spec(dw2.shape), _full_spec(db2.shape),
            _full_spec(dw3.shape), _full_spec(db3.shape),
        ],
        out_specs=pl.BlockSpec((1, M, 1), lambda b: (b, 0, 0)),
        out_shape=jax.ShapeDtypeStruct((B, M, 1), F32),
    )(xt, pos, lat, qw1, qb1, qw2, qb2, qw3, qb3,
      dw1, db1, dw2, db2, dw3, db3)


def _pad_rows(w):
    return jnp.concatenate(
        [w, jnp.zeros((PAD - IN_DIM, w.shape[1]), F32)], axis=0)


def kernel(xc, yc, xt, params, senders, receivers):
    pos = params['pos']
    blk = params['block']
    B = xc.shape[0]

    wm_r = _pad_rows(blk['Wm'][:IN_DIM])
    wm_s = _pad_rows(blk['Wm'][IN_DIM:])
    wn_r = _pad_rows(blk['Wn'][:IN_DIM])
    wn_s = _pad_rows(blk['Wn'][IN_DIM:])

    nodes = _encode(xc, yc, pos, params['enc'])
    zeros = jnp.zeros((N_NODES, PAD), F32)

    lat = None
    for _ in range(3):
        gr, gs = _sc_gather2(nodes, receivers, senders)
        msgs = _messages(gr, gs, wm_r, wm_s, blk['bm'], blk['g1'], blk['b1'])
        inbox = jnp.stack(
            [_sc_scatter(msgs[b], receivers, zeros) for b in range(B)])
        nodes, lat = _update(nodes, inbox, pos, wn_r, wn_s,
                             blk['bn'], blk['g2'], blk['b2'])

    return _decode(xt, pos, lat, params['qenc'], params['dec'])
